# baseline (speedup 1.0000x reference)
import functools

import jax
import jax.numpy as jnp
from jax import lax
from jax.experimental import pallas as pl
from jax.experimental.pallas import tpu as pltpu
from jax.experimental.pallas import tpu_sc as plsc

N = 10000
D = 128
G = 64
NC = 2
NS = 16
NW = NC * NS
CHUNK = 128
NPAD = 10112
STRIPE = NPAD // NS


def _make_agg(n_chunks):
    mesh = plsc.VectorSubcoreMesh(
        core_axis_name="c", subcore_axis_name="s",
        num_cores=NC, num_subcores=NS)

    @functools.partial(
        pl.kernel,
        out_type=jax.ShapeDtypeStruct((NC, NPAD, D), jnp.float32),
        mesh=mesh,
        scratch_types=[
            pltpu.VMEM((CHUNK,), jnp.int32),
            pltpu.VMEM((CHUNK,), jnp.int32),
            pltpu.VMEM((CHUNK, D), jnp.float32),
            pltpu.VMEM_SHARED((NPAD, D), jnp.float32),
            pltpu.SemaphoreType.DMA,
        ],
    )
    def agg(h_hbm, src_hbm, dst_hbm, zeros_hbm, out_hbm,
            src_a, dst_a, rows_v, agg_sh, gsem):
        c = lax.axis_index("c")
        s = lax.axis_index("s")
        row0 = s * STRIPE
        pltpu.sync_copy(zeros_hbm.at[pl.ds(row0, STRIPE)],
                        agg_sh.at[pl.ds(row0, STRIPE)])
        plsc.subcore_barrier()

        wid = s * NC + c
        base0 = wid * n_chunks * CHUNK

        def body(j, carry):
            base = pl.multiple_of(base0 + j * CHUNK, CHUNK)
            pltpu.sync_copy(src_hbm.at[pl.ds(base, CHUNK)], src_a)
            pltpu.sync_copy(dst_hbm.at[pl.ds(base, CHUNK)], dst_a)
            pltpu.async_copy(h_hbm.at[src_a], rows_v, gsem).wait()
            pltpu.sync_copy(rows_v, agg_sh.at[dst_a], add=True)
            return carry

        lax.fori_loop(0, n_chunks, body, 0)
        plsc.subcore_barrier()
        pltpu.sync_copy(agg_sh.at[pl.ds(row0, STRIPE)],
                        out_hbm.at[c, pl.ds(row0, STRIPE)])

    return agg


def _mlp_body(h_ref, pa_ref, w1_ref, b1_ref, g_ref, be_ref,
              w2_ref, b2_ref, o_ref):
    z = h_ref[...] + pa_ref[0, :N] + pa_ref[1, :N]
    a = jnp.dot(z, w1_ref[...], preferred_element_type=jnp.float32) + b1_ref[...]
    mean = jnp.mean(a, axis=0, keepdims=True)
    cent = a - mean
    var = jnp.mean(cent * cent, axis=0, keepdims=True)
    an = g_ref[...] * cent * lax.rsqrt(var + 1e-5) + be_ref[...]
    r = jnp.maximum(an, 0.0)
    o_ref[...] = jnp.maximum(
        jnp.dot(r, w2_ref[...], preferred_element_type=jnp.float32) + b2_ref[...],
        0.0)


def _final_body(h_ref, pa_ref, w1_ref, b1_ref, g_ref, be_ref,
                w2_ref, b2_ref, batch_ref, hw1_ref, hb1_ref, hw2_ref,
                hb2_ref, o_ref):
    z = h_ref[...] + pa_ref[0, :N] + pa_ref[1, :N]
    a = jnp.dot(z, w1_ref[...], preferred_element_type=jnp.float32) + b1_ref[...]
    mean = jnp.mean(a, axis=0, keepdims=True)
    cent = a - mean
    var = jnp.mean(cent * cent, axis=0, keepdims=True)
    an = g_ref[...] * cent * lax.rsqrt(var + 1e-5) + be_ref[...]
    r = jnp.maximum(an, 0.0)
    h2 = jnp.maximum(
        jnp.dot(r, w2_ref[...], preferred_element_type=jnp.float32) + b2_ref[...],
        0.0)
    gids = lax.broadcasted_iota(jnp.int32, (G, N), 0)
    onehot = (gids == batch_ref[...]).astype(jnp.float32)
    pooled = jnp.dot(onehot, h2, preferred_element_type=jnp.float32)
    t = jnp.maximum(
        jnp.dot(pooled, hw1_ref[...], preferred_element_type=jnp.float32)
        + hb1_ref[...], 0.0)
    o_ref[...] = (jnp.dot(t, hw2_ref[...], preferred_element_type=jnp.float32)
                  + hb2_ref[...])


def _mlp_call(h, pa, w1, b1, g, be, w2, b2):
    return pl.pallas_call(
        _mlp_body,
        out_shape=jax.ShapeDtypeStruct((N, D), jnp.float32),
    )(h, pa, w1, b1.reshape(1, -1), g.reshape(1, -1),
      be.reshape(1, -1), w2, b2.reshape(1, -1))


def _final_call(h, pa, w1, b1, g, be, w2, b2, batch, hw1, hb1, hw2, hb2):
    return pl.pallas_call(
        _final_body,
        out_shape=jax.ShapeDtypeStruct((G, hw2.shape[1]), jnp.float32),
    )(h, pa, w1, b1.reshape(1, -1), g.reshape(1, -1),
      be.reshape(1, -1), w2, b2.reshape(1, -1),
      batch.astype(jnp.int32).reshape(1, -1),
      hw1, hb1.reshape(1, -1), hw2, hb2.reshape(1, -1))


def kernel(x, edge_index, batch,
           l0_W1, l0_b1, l0_gamma, l0_beta, l0_W2, l0_b2,
           l1_W1, l1_b1, l1_gamma, l1_beta, l1_W2, l1_b2,
           l2_W1, l2_b1, l2_gamma, l2_beta, l2_W2, l2_b2,
           head_W1, head_b1, head_W2, head_b2):
    src = edge_index[0].astype(jnp.int32)
    dst = edge_index[1].astype(jnp.int32)
    e = src.shape[0]
    per = NW * CHUNK
    e_pad = ((e + per - 1) // per) * per
    n_chunks = e_pad // (NW * CHUNK)
    pad = e_pad - e
    if pad:
        src = jnp.concatenate([src, jnp.zeros((pad,), jnp.int32)])
        dst = jnp.concatenate([dst, jnp.full((pad,), N, jnp.int32)])
    zeros = jnp.zeros((NPAD, D), jnp.float32)
    agg_fn = _make_agg(n_chunks)

    layers = [
        (l0_W1, l0_b1, l0_gamma, l0_beta, l0_W2, l0_b2),
        (l1_W1, l1_b1, l1_gamma, l1_beta, l1_W2, l1_b2),
        (l2_W1, l2_b1, l2_gamma, l2_beta, l2_W2, l2_b2),
    ]
    h = x
    for layer in range(2):
        w1, b1, g, be, w2, b2 = layers[layer]
        pa = agg_fn(h, src, dst, zeros)
        h = _mlp_call(h, pa, w1, b1, g, be, w2, b2)
    w1, b1, g, be, w2, b2 = layers[2]
    pa = agg_fn(h, src, dst, zeros)
    return _final_call(h, pa, w1, b1, g, be, w2, b2,
                       batch, head_W1, head_b1, head_W2, head_b2)


# speedup vs baseline: 1.0790x; 1.0790x over previous
import functools

import jax
import jax.numpy as jnp
from jax import lax
from jax.experimental import pallas as pl
from jax.experimental.pallas import tpu as pltpu
from jax.experimental.pallas import tpu_sc as plsc

N = 10000
D = 128
G = 64
NC = 2
NS = 16
NW = NC * NS
CHUNK = 128
NPAD = 10112
STRIPE = NPAD // NS


def _make_agg(n_chunks):
    mesh = plsc.VectorSubcoreMesh(
        core_axis_name="c", subcore_axis_name="s",
        num_cores=NC, num_subcores=NS)

    @functools.partial(
        pl.kernel,
        out_type=jax.ShapeDtypeStruct((NC, NPAD, D), jnp.float32),
        mesh=mesh,
        scratch_types=[
            pltpu.VMEM((2, CHUNK), jnp.int32),
            pltpu.VMEM((CHUNK, D), jnp.float32),
            pltpu.VMEM_SHARED((NPAD, D), jnp.float32),
            pltpu.SemaphoreType.DMA,
        ],
    )
    def agg(h_hbm, idx_hbm, zeros_hbm, out_hbm,
            idx_v, rows_v, agg_sh, gsem):
        c = lax.axis_index("c")
        s = lax.axis_index("s")
        row0 = s * STRIPE
        pltpu.sync_copy(zeros_hbm.at[pl.ds(row0, STRIPE)],
                        agg_sh.at[pl.ds(row0, STRIPE)])
        plsc.subcore_barrier()

        wid = s * NC + c
        base0 = wid * n_chunks

        def body(j, carry):
            pltpu.sync_copy(idx_hbm.at[base0 + j], idx_v)
            pltpu.async_copy(h_hbm.at[idx_v.at[0]], rows_v, gsem).wait()
            pltpu.sync_copy(rows_v, agg_sh.at[idx_v.at[1]], add=True)
            return carry

        lax.fori_loop(0, n_chunks, body, 0)
        plsc.subcore_barrier()
        pltpu.sync_copy(agg_sh.at[pl.ds(row0, STRIPE)],
                        out_hbm.at[c, pl.ds(row0, STRIPE)])

    return agg


def _mlp_body(h_ref, pa_ref, w1_ref, b1_ref, g_ref, be_ref,
              w2_ref, b2_ref, o_ref):
    z = h_ref[...] + pa_ref[0, :N] + pa_ref[1, :N]
    a = jnp.dot(z, w1_ref[...], preferred_element_type=jnp.float32) + b1_ref[...]
    mean = jnp.mean(a, axis=0, keepdims=True)
    cent = a - mean
    var = jnp.mean(cent * cent, axis=0, keepdims=True)
    an = g_ref[...] * cent * lax.rsqrt(var + 1e-5) + be_ref[...]
    r = jnp.maximum(an, 0.0)
    o_ref[...] = jnp.maximum(
        jnp.dot(r, w2_ref[...], preferred_element_type=jnp.float32) + b2_ref[...],
        0.0)


def _final_body(h_ref, pa_ref, w1_ref, b1_ref, g_ref, be_ref,
                w2_ref, b2_ref, batch_ref, hw1_ref, hb1_ref, hw2_ref,
                hb2_ref, o_ref):
    z = h_ref[...] + pa_ref[0, :N] + pa_ref[1, :N]
    a = jnp.dot(z, w1_ref[...], preferred_element_type=jnp.float32) + b1_ref[...]
    mean = jnp.mean(a, axis=0, keepdims=True)
    cent = a - mean
    var = jnp.mean(cent * cent, axis=0, keepdims=True)
    an = g_ref[...] * cent * lax.rsqrt(var + 1e-5) + be_ref[...]
    r = jnp.maximum(an, 0.0)
    h2 = jnp.maximum(
        jnp.dot(r, w2_ref[...], preferred_element_type=jnp.float32) + b2_ref[...],
        0.0)
    gids = lax.broadcasted_iota(jnp.int32, (G, N), 0)
    onehot = (gids == batch_ref[...]).astype(jnp.float32)
    pooled = jnp.dot(onehot, h2, preferred_element_type=jnp.float32)
    t = jnp.maximum(
        jnp.dot(pooled, hw1_ref[...], preferred_element_type=jnp.float32)
        + hb1_ref[...], 0.0)
    o_ref[...] = (jnp.dot(t, hw2_ref[...], preferred_element_type=jnp.float32)
                  + hb2_ref[...])


def _mlp_call(h, pa, w1, b1, g, be, w2, b2):
    return pl.pallas_call(
        _mlp_body,
        out_shape=jax.ShapeDtypeStruct((N, D), jnp.float32),
    )(h, pa, w1, b1.reshape(1, -1), g.reshape(1, -1),
      be.reshape(1, -1), w2, b2.reshape(1, -1))


def _final_call(h, pa, w1, b1, g, be, w2, b2, batch, hw1, hb1, hw2, hb2):
    return pl.pallas_call(
        _final_body,
        out_shape=jax.ShapeDtypeStruct((G, hw2.shape[1]), jnp.float32),
    )(h, pa, w1, b1.reshape(1, -1), g.reshape(1, -1),
      be.reshape(1, -1), w2, b2.reshape(1, -1),
      batch.astype(jnp.int32).reshape(1, -1),
      hw1, hb1.reshape(1, -1), hw2, hb2.reshape(1, -1))


def kernel(x, edge_index, batch,
           l0_W1, l0_b1, l0_gamma, l0_beta, l0_W2, l0_b2,
           l1_W1, l1_b1, l1_gamma, l1_beta, l1_W2, l1_b2,
           l2_W1, l2_b1, l2_gamma, l2_beta, l2_W2, l2_b2,
           head_W1, head_b1, head_W2, head_b2):
    src = edge_index[0].astype(jnp.int32)
    dst = edge_index[1].astype(jnp.int32)
    e = src.shape[0]
    per = NW * CHUNK
    e_pad = ((e + per - 1) // per) * per
    n_chunks = e_pad // (NW * CHUNK)
    pad = e_pad - e
    if pad:
        src = jnp.concatenate([src, jnp.zeros((pad,), jnp.int32)])
        dst = jnp.concatenate([dst, jnp.full((pad,), N, jnp.int32)])
    idx = jnp.stack([src.reshape(-1, CHUNK), dst.reshape(-1, CHUNK)], axis=1)
    zeros = jnp.zeros((NPAD, D), jnp.float32)
    agg_fn = _make_agg(n_chunks)

    layers = [
        (l0_W1, l0_b1, l0_gamma, l0_beta, l0_W2, l0_b2),
        (l1_W1, l1_b1, l1_gamma, l1_beta, l1_W2, l1_b2),
        (l2_W1, l2_b1, l2_gamma, l2_beta, l2_W2, l2_b2),
    ]
    h = x
    for layer in range(2):
        w1, b1, g, be, w2, b2 = layers[layer]
        pa = agg_fn(h, idx, zeros)
        h = _mlp_call(h, pa, w1, b1, g, be, w2, b2)
    w1, b1, g, be, w2, b2 = layers[2]
    pa = agg_fn(h, idx, zeros)
    return _final_call(h, pa, w1, b1, g, be, w2, b2,
                       batch, head_W1, head_b1, head_W2, head_b2)
